# baseline (speedup 1.0000x reference)
import functools

import jax
import jax.numpy as jnp
from jax import lax
from jax.experimental import pallas as pl
from jax.experimental.pallas import tpu as pltpu
from jax.experimental.pallas import tpu_sc as plsc

_NC, _NS, _L = 2, 16, 16
_NW = _NC * _NS
_R = 32
_G = 8
_NBUF = 3


def _rsqrt16(ssv):
    i = plsc.bitcast(ssv, jnp.int32)
    i = jnp.int32(0x5F3759DF) - (i >> 1)
    y = plsc.bitcast(i, jnp.float32)
    for _ in range(2):
        y = y * (1.5 - 0.5 * ssv * y * y)
    return y


def _sc_call(x, b):
    n, d = x.shape
    rows_per_w = n // _NW
    nblk = rows_per_w // _R
    nch = d // _L
    mesh = plsc.VectorSubcoreMesh(
        core_axis_name="c", subcore_axis_name="s",
        num_cores=_NC, num_subcores=_NS)

    @functools.partial(
        pl.kernel,
        out_type=jax.ShapeDtypeStruct((n, d), jnp.float32),
        mesh=mesh,
        compiler_params=pltpu.CompilerParams(needs_layout_passes=False),
        scratch_types=[
            pltpu.VMEM((d,), jnp.float32),
            pltpu.VMEM((_NBUF, _R, d), jnp.float32),
            pltpu.SemaphoreType.DMA,
            pltpu.SemaphoreType.DMA,
            pltpu.SemaphoreType.DMA,
            pltpu.SemaphoreType.DMA,
            pltpu.SemaphoreType.DMA,
            pltpu.SemaphoreType.DMA,
        ],
    )
    def k(x_hbm, b_hbm, out_hbm, b_v, buf,
          isem0, isem1, isem2, osem0, osem1, osem2):
        wid = lax.axis_index("s") * _NC + lax.axis_index("c")
        pltpu.sync_copy(b_hbm, b_v)
        base = wid * rows_per_w
        isem = (isem0, isem1, isem2)
        osem = (osem0, osem1, osem2)

        def start_in(j, p):
            pltpu.async_copy(x_hbm.at[pl.ds(base + j * _R, _R)], buf.at[p], isem[p])

        def wait_in(j, p):
            pltpu.make_async_copy(
                x_hbm.at[pl.ds(base + j * _R, _R)], buf.at[p], isem[p]).wait()

        def start_out(j, p):
            pltpu.async_copy(buf.at[p], out_hbm.at[pl.ds(base + j * _R, _R)], osem[p])

        def wait_out(j, p):
            pltpu.make_async_copy(
                buf.at[p], out_hbm.at[pl.ds(base + j * _R, _R)], osem[p]).wait()

        def start_out_half(j, p, h):
            hr = _R // 2
            pltpu.async_copy(
                buf.at[p, pl.ds(h * hr, hr)],
                out_hbm.at[pl.ds(base + j * _R + h * hr, hr)], osem[p])

        def compute_half(p, h):
            def group(g, carry):
                zero = jnp.zeros((_L,), jnp.float32)

                @plsc.parallel_loop(0, nch, unroll=4, carry=(zero,) * (2 * _G))
                def acc(j, c):
                    a = b_v[pl.ds(j * _L, _L)]
                    out = []
                    for t in range(_G):
                        v = buf[p, g * _G + t, pl.ds(j * _L, _L)]
                        out.append(c[2 * t] + v * v)
                        out.append(c[2 * t + 1] + v * a)
                    return tuple(out)

                coefs = []
                for t in range(_G):
                    ss = jnp.sum(acc[2 * t])
                    dd = jnp.sum(acc[2 * t + 1])
                    ssv = jnp.full((_L,), ss)
                    normv = ssv * _rsqrt16(ssv)
                    coefs.append(dd / jnp.maximum(normv, 1e-12))

                @plsc.parallel_loop(0, nch, unroll=4)
                def _(j):
                    a = b_v[pl.ds(j * _L, _L)]
                    for t in range(_G):
                        v = buf[p, g * _G + t, pl.ds(j * _L, _L)]
                        buf[p, g * _G + t, pl.ds(j * _L, _L)] = v + coefs[t] * a

                return carry

            hg = _R // _G // 2
            lax.fori_loop(h * hg, (h + 1) * hg, group, 0)

        def compute(p):
            compute_half(p, 0)
            compute_half(p, 1)

        start_in(0, 0)
        start_in(1, 1)
        niter = nblk // _NBUF

        def blk_iter(i, carry):
            for jj in range(_NBUF):
                j = i * _NBUF + jj
                p = jj

                wait_in(j, p)
                compute_half(p, 0)
                start_out_half(j, p, 0)
                compute_half(p, 1)

                @pl.when(jax.lax.ge(j, 1) & jax.lax.lt(j + 2, nblk))
                def _drain():
                    wait_out(j - 1, (jj - 1) % _NBUF)

                @pl.when(jax.lax.lt(j + 2, nblk))
                def _prefetch():
                    start_in(j + 2, (jj + 2) % _NBUF)

                start_out_half(j, p, 1)
            return carry

        lax.fori_loop(0, niter, blk_iter, 0)
        for j in range(_NBUF * (nblk // _NBUF), nblk):
            p = j % _NBUF
            wait_in(j, p)
            compute(p)
            start_out(j, p)
        for j in range(nblk - 3, nblk):
            wait_out(j, j % _NBUF)

    return k(x, b)


def kernel(x, axes, strength):
    alpha = 0.5
    s = jax.nn.relu(strength[:1]) + 1e-6
    w = s / jnp.sum(s)
    b = jnp.sqrt(alpha * w[0]) * axes[0]
    return _sc_call(x, b)


# speedup vs baseline: 1.1341x; 1.1341x over previous
import functools

import jax
import jax.numpy as jnp
from jax import lax
from jax.experimental import pallas as pl
from jax.experimental.pallas import tpu as pltpu
from jax.experimental.pallas import tpu_sc as plsc

_NC, _NS, _L = 2, 16, 16
_NW = _NC * _NS
_R = 32
_G = 8
_NBUF = 3


def _rsqrt16(ssv):
    i = plsc.bitcast(ssv, jnp.int32)
    i = jnp.int32(0x5F3759DF) - (i >> 1)
    y = plsc.bitcast(i, jnp.float32)
    for _ in range(2):
        y = y * (1.5 - 0.5 * ssv * y * y)
    return y


def _sc_call(x, b):
    n, d = x.shape
    rows_per_w = n // _NW
    nblk = rows_per_w // _R
    nch = d // _L
    mesh = plsc.VectorSubcoreMesh(
        core_axis_name="c", subcore_axis_name="s",
        num_cores=_NC, num_subcores=_NS)

    @functools.partial(
        pl.kernel,
        out_type=jax.ShapeDtypeStruct((n, d), jnp.float32),
        mesh=mesh,
        compiler_params=pltpu.CompilerParams(needs_layout_passes=False),
        scratch_types=[
            pltpu.VMEM((d,), jnp.float32),
            pltpu.VMEM((d,), jnp.bfloat16),
            pltpu.VMEM((_NBUF, _R, d), jnp.float32),
            pltpu.SemaphoreType.DMA,
            pltpu.SemaphoreType.DMA,
            pltpu.SemaphoreType.DMA,
            pltpu.SemaphoreType.DMA,
            pltpu.SemaphoreType.DMA,
            pltpu.SemaphoreType.DMA,
        ],
    )
    def k(x_hbm, b_hbm, out_hbm, b_v, bb, buf,
          isem0, isem1, isem2, osem0, osem1, osem2):
        wid = lax.axis_index("s") * _NC + lax.axis_index("c")
        pltpu.sync_copy(b_hbm, b_v)

        @plsc.parallel_loop(0, nch // 2, unroll=4)
        def _pack_b(i):
            b0 = b_v[pl.ds(i * 2 * _L, _L)]
            b1 = b_v[pl.ds(i * 2 * _L + _L, _L)]
            bb[pl.ds(i * 2 * _L, 2 * _L)] = plsc.pack(
                b0, b1, format=plsc.PackFormat.INTERLEAVED)

        base = wid * rows_per_w
        isem = (isem0, isem1, isem2)
        osem = (osem0, osem1, osem2)

        def start_in(j, p):
            pltpu.async_copy(x_hbm.at[pl.ds(base + j * _R, _R)], buf.at[p], isem[p])

        def wait_in(j, p):
            pltpu.make_async_copy(
                x_hbm.at[pl.ds(base + j * _R, _R)], buf.at[p], isem[p]).wait()

        def start_out(j, p):
            pltpu.async_copy(buf.at[p], out_hbm.at[pl.ds(base + j * _R, _R)], osem[p])

        def wait_out(j, p):
            pltpu.make_async_copy(
                buf.at[p], out_hbm.at[pl.ds(base + j * _R, _R)], osem[p]).wait()

        def compute(p):
            def group(g, carry):
                zero = jnp.zeros((2 * _L,), jnp.bfloat16)

                @plsc.parallel_loop(0, nch // 2, unroll=4, carry=(zero,) * (2 * _G))
                def acc(j, c):
                    ab = bb[pl.ds(j * 2 * _L, 2 * _L)]
                    out = []
                    for t in range(_G):
                        v0 = buf[p, g * _G + t, pl.ds(j * 2 * _L, _L)]
                        v1 = buf[p, g * _G + t, pl.ds(j * 2 * _L + _L, _L)]
                        vb = plsc.pack(v0, v1, format=plsc.PackFormat.INTERLEAVED)
                        out.append(c[2 * t] + vb * vb)
                        out.append(c[2 * t + 1] + vb * ab)
                    return tuple(out)

                coefs = []
                for t in range(_G):
                    sq0, sq1 = plsc.unpack(
                        acc[2 * t], format=plsc.PackFormat.INTERLEAVED)
                    dt0, dt1 = plsc.unpack(
                        acc[2 * t + 1], format=plsc.PackFormat.INTERLEAVED)
                    ss = jnp.sum(sq0 + sq1)
                    dd = jnp.sum(dt0 + dt1)
                    ssv = jnp.full((_L,), ss)
                    normv = ssv * _rsqrt16(ssv)
                    coefs.append(dd / jnp.maximum(normv, 1e-12))

                @plsc.parallel_loop(0, nch, unroll=4)
                def _(j):
                    a = b_v[pl.ds(j * _L, _L)]
                    for t in range(_G):
                        v = buf[p, g * _G + t, pl.ds(j * _L, _L)]
                        buf[p, g * _G + t, pl.ds(j * _L, _L)] = v + coefs[t] * a

                return carry

            lax.fori_loop(0, _R // _G, group, 0)

        start_in(0, 0)
        start_in(1, 1)
        niter = nblk // _NBUF

        def blk_iter(i, carry):
            for jj in range(_NBUF):
                j = i * _NBUF + jj
                p = jj

                wait_in(j, p)
                compute(p)

                @pl.when(jax.lax.ge(j, 1) & jax.lax.lt(j + 2, nblk))
                def _drain():
                    wait_out(j - 1, (jj - 1) % _NBUF)

                @pl.when(jax.lax.lt(j + 2, nblk))
                def _prefetch():
                    start_in(j + 2, (jj + 2) % _NBUF)

                start_out(j, p)
            return carry

        lax.fori_loop(0, niter, blk_iter, 0)
        for j in range(_NBUF * (nblk // _NBUF), nblk):
            p = j % _NBUF
            wait_in(j, p)
            compute(p)
            start_out(j, p)
        for j in range(nblk - 3, nblk):
            wait_out(j, j % _NBUF)

    return k(x, b)


def kernel(x, axes, strength):
    alpha = 0.5
    s = jax.nn.relu(strength[:1]) + 1e-6
    w = s / jnp.sum(s)
    b = jnp.sqrt(alpha * w[0]) * axes[0]
    return _sc_call(x, b)
